# Initial kernel scaffold; baseline (speedup 1.0000x reference)
import jax
import jax.numpy as jnp
from jax.experimental import pallas as pl


def kernel(x, edge_index, edge_type, weight, root, bias):
    raise NotImplementedError("write your pallas kernel here")


# speedup vs baseline: 4.0300x; 4.0300x over previous
import functools

import jax
import jax.numpy as jnp
from jax import lax
from jax.experimental import pallas as pl
from jax.experimental.pallas import tpu as pltpu
from jax.experimental.pallas import tpu_sc as plsc

N = 10000
E = 320000
D = 128
OUT = 128
R = 8
NR = N * R

NC = 2
NS = 16
NW = NC * NS
EPW = E // NW
K = 80
NCHUNK = EPW // K
SEG_PW = NR // NS
DRAIN = 200
NDRAIN = N // DRAIN

_mesh = plsc.VectorSubcoreMesh(core_axis_name="c", subcore_axis_name="s")


@functools.partial(
    pl.kernel,
    mesh=_mesh,
    out_type=jax.ShapeDtypeStruct((NC * NR,), jnp.float32),
    scratch_types=[
        pltpu.VMEM((K,), jnp.int32),
        pltpu.VMEM((K,), jnp.int32),
        pltpu.VMEM((K,), jnp.int32),
        pltpu.VMEM((K,), jnp.float32),
        pltpu.VMEM((SEG_PW,), jnp.float32),
        pltpu.VMEM_SHARED((NR,), jnp.float32),
    ],
)
def _count_kernel(dst_hbm, et_hbm, cnt_hbm, dstb, etb, segb, onesb, stage,
                  cnt_sh):
    c = lax.axis_index("c")
    s = lax.axis_index("s")
    w = s * NC + c

    ones16 = jnp.full((16,), 1.0, jnp.float32)
    zeros16 = jnp.zeros((16,), jnp.float32)
    for g in range(K // 16):
        onesb[pl.ds(g * 16, 16)] = ones16

    def _z(i, _):
        stage[pl.ds(i * 16, 16)] = zeros16
        return 0
    lax.fori_loop(0, SEG_PW // 16, _z, 0)
    pltpu.sync_copy(stage, cnt_sh.at[pl.ds(s * SEG_PW, SEG_PW)])
    plsc.subcore_barrier()

    def _chunk(i, _):
        base = w * EPW + i * K
        pltpu.sync_copy(dst_hbm.at[pl.ds(base, K)], dstb)
        pltpu.sync_copy(et_hbm.at[pl.ds(base, K)], etb)
        for g in range(K // 16):
            d16 = dstb[pl.ds(g * 16, 16)]
            e16 = etb[pl.ds(g * 16, 16)]
            segb[pl.ds(g * 16, 16)] = d16 * R + e16
        pltpu.sync_copy(onesb, cnt_sh.at[segb], add=True)
        return 0
    lax.fori_loop(0, NCHUNK, _chunk, 0)

    plsc.subcore_barrier()
    pltpu.sync_copy(cnt_sh.at[pl.ds(s * SEG_PW, SEG_PW)], stage)
    pltpu.sync_copy(stage, cnt_hbm.at[pl.ds(c * NR + s * SEG_PW, SEG_PW)])


@functools.partial(
    pl.kernel,
    mesh=_mesh,
    out_type=jax.ShapeDtypeStruct((NC, N, OUT), jnp.float32),
    scratch_types=[
        pltpu.VMEM((K,), jnp.int32),
        pltpu.VMEM((K,), jnp.int32),
        pltpu.VMEM((K,), jnp.int32),
        pltpu.VMEM((K,), jnp.int32),
        pltpu.VMEM((K,), jnp.int32),
        pltpu.VMEM((K + 16,), jnp.float32),
        pltpu.VMEM((K, OUT), jnp.float32),
        pltpu.VMEM((DRAIN, OUT), jnp.float32),
        pltpu.SemaphoreType.DMA,
        pltpu.SemaphoreType.DMA,
        pltpu.VMEM_SHARED((N, OUT), jnp.float32),
    ],
)
def _scatter_kernel(src_hbm, dst_hbm, et_hbm, y_hbm, inv_hbm, part_hbm,
                    srcb, dstb, etb, gidxb, segb, inveb, rows, zst, sem,
                    sem2, acc_sh):
    c = lax.axis_index("c")
    s = lax.axis_index("s")
    w = s * NC + c

    zeros16 = jnp.zeros((16,), jnp.float32)

    def _z(i, _):
        for k in range(OUT // 16):
            zst[i, pl.ds(k * 16, 16)] = zeros16
        return 0
    lax.fori_loop(0, DRAIN, _z, 0)
    for j in range((NDRAIN + NS - 1) // NS):
        idx = s + j * NS

        @pl.when(idx < NDRAIN)
        def _():
            pltpu.sync_copy(zst, acc_sh.at[pl.ds(idx * DRAIN, DRAIN), :])
    plsc.subcore_barrier()

    def _chunk(i, _):
        base = w * EPW + i * K
        pltpu.sync_copy(src_hbm.at[pl.ds(base, K)], srcb)
        pltpu.sync_copy(dst_hbm.at[pl.ds(base, K)], dstb)
        pltpu.sync_copy(et_hbm.at[pl.ds(base, K)], etb)
        for g in range(K // 16):
            s16 = srcb[pl.ds(g * 16, 16)]
            d16 = dstb[pl.ds(g * 16, 16)]
            e16 = etb[pl.ds(g * 16, 16)]
            gidxb[pl.ds(g * 16, 16)] = s16 * R + e16
            segb[pl.ds(g * 16, 16)] = d16 * R + e16
        cp1 = pltpu.async_copy(y_hbm.at[gidxb], rows, sem)
        cp2 = pltpu.async_copy(inv_hbm.at[segb], inveb.at[pl.ds(0, K)], sem2)
        cp1.wait()
        cp2.wait()

        def _scale(j, _):
            v = inveb[pl.ds(j, 16)][0]
            for k in range(OUT // 16):
                rows[j, pl.ds(k * 16, 16)] = rows[j, pl.ds(k * 16, 16)] * v
            return 0
        lax.fori_loop(0, K, _scale, 0)

        pltpu.sync_copy(rows, acc_sh.at[dstb], add=True)
        return 0
    lax.fori_loop(0, NCHUNK, _chunk, 0)

    plsc.subcore_barrier()
    for j in range((NDRAIN + NS - 1) // NS):
        idx = s + j * NS

        @pl.when(idx < NDRAIN)
        def _():
            r0 = idx * DRAIN
            pltpu.sync_copy(acc_sh.at[pl.ds(r0, DRAIN), :], zst)
            pltpu.sync_copy(zst, part_hbm.at[c, pl.ds(r0, DRAIN), :])


def _y_body(x_ref, w_ref, o_ref):
    o_ref[...] = jnp.dot(x_ref[...], w_ref[...],
                         preferred_element_type=jnp.float32)


def _inv_body(cnt_ref, o_ref):
    total = cnt_ref[0] + cnt_ref[1]
    o_ref[...] = 1.0 / jnp.maximum(total, 1.0)


def _final_body(p_ref, x_ref, r_ref, b_ref, o_ref):
    o_ref[...] = (p_ref[0] + p_ref[1]
                  + jnp.dot(x_ref[...], r_ref[...],
                            preferred_element_type=jnp.float32)
                  + b_ref[...])


_BN = 1000


def kernel(x, edge_index, edge_type, weight, root, bias):
    src = edge_index[0]
    dst = edge_index[1]
    et = edge_type

    wf = weight.transpose(1, 0, 2).reshape(D, R * OUT)
    y = pl.pallas_call(
        _y_body,
        grid=(N // _BN,),
        in_specs=[
            pl.BlockSpec((_BN, D), lambda i: (i, 0)),
            pl.BlockSpec((D, R * OUT), lambda i: (0, 0)),
        ],
        out_specs=pl.BlockSpec((_BN, R * OUT), lambda i: (i, 0)),
        out_shape=jax.ShapeDtypeStruct((N, R * OUT), jnp.float32),
    )(x, wf)
    y = y.reshape(NR, OUT)

    cnt_parts = _count_kernel(dst, et)
    inv = pl.pallas_call(
        _inv_body,
        in_specs=[pl.BlockSpec((NC, NR // 128, 128), lambda: (0, 0, 0))],
        out_specs=pl.BlockSpec((NR // 128, 128), lambda: (0, 0)),
        out_shape=jax.ShapeDtypeStruct((NR // 128, 128), jnp.float32),
    )(cnt_parts.reshape(NC, NR // 128, 128))
    inv = inv.reshape(NR)

    parts = _scatter_kernel(src, dst, et, y, inv)

    out = pl.pallas_call(
        _final_body,
        grid=(N // _BN,),
        in_specs=[
            pl.BlockSpec((NC, _BN, OUT), lambda i: (0, i, 0)),
            pl.BlockSpec((_BN, D), lambda i: (i, 0)),
            pl.BlockSpec((D, OUT), lambda i: (0, 0)),
            pl.BlockSpec((1, OUT), lambda i: (0, 0)),
        ],
        out_specs=pl.BlockSpec((_BN, OUT), lambda i: (i, 0)),
        out_shape=jax.ShapeDtypeStruct((N, OUT), jnp.float32),
    )(parts, x, root, bias.reshape(1, OUT))
    return out
